# baseline (speedup 1.0000x reference)
import functools

import jax
import jax.numpy as jnp
from jax import lax
from jax.experimental import pallas as pl
from jax.experimental.pallas import tpu as pltpu
from jax.experimental.pallas import tpu_sc as plsc

_NUM_BOXES = 20000
_NUM_CLASSES = 80
_NUM_SEL = 20000


_BLK = 2048
_GRID = (_NUM_BOXES + _BLK - 1) // _BLK


def _reduce_body(s_ref, maxv_ref, cls_ref):
    s = s_ref[...]
    m = jnp.max(s, axis=0, keepdims=True)
    ids = lax.broadcasted_iota(jnp.int32, s.shape, 0)
    cl = jnp.min(jnp.where(s == m, ids, _NUM_CLASSES), axis=0, keepdims=True)
    maxv_ref[...] = m
    cls_ref[...] = cl.astype(jnp.float32)


def _cls_reduce(scores2d):
    return pl.pallas_call(
        _reduce_body,
        grid=(_GRID,),
        in_specs=[pl.BlockSpec((_NUM_CLASSES, _BLK), lambda i: (0, i))],
        out_specs=[
            pl.BlockSpec((1, _BLK), lambda i: (0, i)),
            pl.BlockSpec((1, _BLK), lambda i: (0, i)),
        ],
        out_shape=[
            jax.ShapeDtypeStruct((1, _NUM_BOXES), jnp.float32),
            jax.ShapeDtypeStruct((1, _NUM_BOXES), jnp.float32),
        ],
    )(scores2d)


_NUM_SC_CORES = 2
_NUM_SC_SUBCORES = 16
_NW = _NUM_SC_CORES * _NUM_SC_SUBCORES
_CHW = 624
_TAIL_BASE = _NW * _CHW
_TAIL = _NUM_SEL - _TAIL_BASE

_mesh = plsc.VectorSubcoreMesh(
    core_axis_name="c", subcore_axis_name="s",
    num_cores=_NUM_SC_CORES, num_subcores=_NUM_SC_SUBCORES,
)


@functools.partial(
    pl.kernel,
    out_type=jax.ShapeDtypeStruct((_NUM_SEL, 6), jnp.float32),
    mesh=_mesh,
    scratch_types=[
        pltpu.VMEM((_CHW,), jnp.int32),
        pltpu.VMEM((_CHW, 8), jnp.float32),
        pltpu.VMEM((_TAIL,), jnp.int32),
        pltpu.VMEM((_TAIL, 8), jnp.float32),
        pltpu.SemaphoreType.DMA,
    ],
    compiler_params=pltpu.CompilerParams(use_tc_tiling_on_sc=False),
)
def _sc_gather(table_hbm, idx_hbm, out_hbm, idx_v, rows_v, tidx_v, trows_v, sem):
    wid = lax.axis_index("s") * _NUM_SC_CORES + lax.axis_index("c")
    base = wid * _CHW
    pltpu.sync_copy(idx_hbm.at[pl.ds(base, _CHW)], idx_v)
    pltpu.async_copy(table_hbm.at[idx_v], rows_v, sem).wait()
    pltpu.sync_copy(rows_v.at[:, pl.ds(0, 6)], out_hbm.at[pl.ds(base, _CHW)])

    @pl.when(wid == 0)
    def _tail():
        pltpu.sync_copy(idx_hbm.at[pl.ds(_TAIL_BASE, _TAIL)], tidx_v)
        pltpu.async_copy(table_hbm.at[tidx_v], trows_v, sem).wait()
        pltpu.sync_copy(trows_v.at[:, pl.ds(0, 6)], out_hbm.at[pl.ds(_TAIL_BASE, _TAIL)])


def kernel(idxTensor, boxes, scores):
    maxv, clsf = _cls_reduce(scores[0])
    table = jnp.stack(
        [boxes[0, 0], boxes[0, 1], boxes[0, 2], boxes[0, 3],
         maxv[0], clsf[0],
         jnp.zeros((_NUM_BOXES,), jnp.float32),
         jnp.zeros((_NUM_BOXES,), jnp.float32)],
        axis=1,
    )
    idx = idxTensor[:, 2]
    concatenated = _sc_gather(table, idx)
    batches = idxTensor[:, 0]
    return (concatenated, batches)


# speedup vs baseline: 2.2821x; 2.2821x over previous
import functools

import jax
import jax.numpy as jnp
from jax import lax
from jax.experimental import pallas as pl
from jax.experimental.pallas import tpu as pltpu
from jax.experimental.pallas import tpu_sc as plsc

_NUM_BOXES = 20000
_NUM_CLASSES = 80
_NUM_SEL = 20000


_BLK = 2048
_GRID = (_NUM_BOXES + _BLK - 1) // _BLK


def _reduce_body(s_ref, maxv_ref, cls_ref):
    s = s_ref[...]
    m = jnp.max(s, axis=0, keepdims=True)
    ids = lax.broadcasted_iota(jnp.int32, s.shape, 0)
    cl = jnp.min(jnp.where(s == m, ids, _NUM_CLASSES), axis=0, keepdims=True)
    maxv_ref[...] = m
    cls_ref[...] = cl.astype(jnp.float32)


def _cls_reduce(scores2d):
    return pl.pallas_call(
        _reduce_body,
        grid=(_GRID,),
        in_specs=[pl.BlockSpec((_NUM_CLASSES, _BLK), lambda i: (0, i))],
        out_specs=[
            pl.BlockSpec((1, _BLK), lambda i: (0, i)),
            pl.BlockSpec((1, _BLK), lambda i: (0, i)),
        ],
        out_shape=[
            jax.ShapeDtypeStruct((1, _NUM_BOXES), jnp.float32),
            jax.ShapeDtypeStruct((1, _NUM_BOXES), jnp.float32),
        ],
    )(scores2d)


_NUM_SC_CORES = 2
_NUM_SC_SUBCORES = 16
_NW = _NUM_SC_CORES * _NUM_SC_SUBCORES
_CHW = 624
_TAIL_BASE = _NW * _CHW
_TAIL = _NUM_SEL - _TAIL_BASE

_mesh = plsc.VectorSubcoreMesh(
    core_axis_name="c", subcore_axis_name="s",
    num_cores=_NUM_SC_CORES, num_subcores=_NUM_SC_SUBCORES,
)

_col_ty = jax.ShapeDtypeStruct((_NUM_SEL,), jnp.float32)


@functools.partial(
    pl.kernel,
    out_type=[_col_ty] * 6,
    mesh=_mesh,
    scratch_types=[
        pltpu.VMEM((_CHW,), jnp.int32),
        pltpu.VMEM((6, _CHW), jnp.float32),
        pltpu.VMEM((_TAIL,), jnp.int32),
        pltpu.VMEM((6, _TAIL), jnp.float32),
        pltpu.SemaphoreType.DMA,
    ],
    compiler_params=pltpu.CompilerParams(use_tc_tiling_on_sc=False),
)
def _sc_gather(b0, b1, b2, b3, mx, cf, idx_hbm,
               o0, o1, o2, o3, o4, o5,
               idx_v, cols_v, tidx_v, tcols_v, sem):
    wid = lax.axis_index("s") * _NUM_SC_CORES + lax.axis_index("c")
    srcs = (b0, b1, b2, b3, mx, cf)
    outs = (o0, o1, o2, o3, o4, o5)

    base = wid * _CHW
    pltpu.sync_copy(idx_hbm.at[pl.ds(base, _CHW)], idx_v)
    copies = [pltpu.async_copy(src.at[idx_v], cols_v.at[j], sem)
              for j, src in enumerate(srcs)]
    for c in copies:
        c.wait()
    for j, out in enumerate(outs):
        pltpu.sync_copy(cols_v.at[j], out.at[pl.ds(base, _CHW)])

    @pl.when(wid == 0)
    def _tail():
        pltpu.sync_copy(idx_hbm.at[pl.ds(_TAIL_BASE, _TAIL)], tidx_v)
        tcopies = [pltpu.async_copy(src.at[tidx_v], tcols_v.at[j], sem)
                   for j, src in enumerate(srcs)]
        for c in tcopies:
            c.wait()
        for j, out in enumerate(outs):
            pltpu.sync_copy(tcols_v.at[j], out.at[pl.ds(_TAIL_BASE, _TAIL)])


def kernel(idxTensor, boxes, scores):
    maxv, clsf = _cls_reduce(scores[0])
    idx = idxTensor[:, 2]
    cols = _sc_gather(
        boxes[0, 0], boxes[0, 1], boxes[0, 2], boxes[0, 3],
        maxv[0], clsf[0], idx,
    )
    concatenated = jnp.stack(cols, axis=1)
    batches = idxTensor[:, 0]
    return (concatenated, batches)


# speedup vs baseline: 2.5086x; 1.0993x over previous
import functools

import jax
import jax.numpy as jnp
from jax import lax
from jax.experimental import pallas as pl
from jax.experimental.pallas import tpu as pltpu
from jax.experimental.pallas import tpu_sc as plsc

_NUM_BOXES = 20000
_NUM_CLASSES = 80
_NUM_SEL = 20000


_BLK = 4096
_GRID = (_NUM_BOXES + _BLK - 1) // _BLK


def _reduce_body(s_ref, maxv_ref, cls_ref):
    s = s_ref[...]
    m = jnp.max(s, axis=0, keepdims=True)
    ids = lax.broadcasted_iota(jnp.int32, s.shape, 0)
    cl = jnp.min(jnp.where(s == m, ids, _NUM_CLASSES), axis=0, keepdims=True)
    maxv_ref[...] = m[0]
    cls_ref[...] = cl.astype(jnp.float32)[0]


def _cls_reduce(scores2d):
    return pl.pallas_call(
        _reduce_body,
        grid=(_GRID,),
        in_specs=[pl.BlockSpec((_NUM_CLASSES, _BLK), lambda i: (0, i))],
        out_specs=[
            pl.BlockSpec((_BLK,), lambda i: (i,)),
            pl.BlockSpec((_BLK,), lambda i: (i,)),
        ],
        out_shape=[
            jax.ShapeDtypeStruct((_NUM_BOXES,), jnp.float32),
            jax.ShapeDtypeStruct((_NUM_BOXES,), jnp.float32),
        ],
    )(scores2d)


_NUM_SC_CORES = 2
_NUM_SC_SUBCORES = 16
_NW = _NUM_SC_CORES * _NUM_SC_SUBCORES
_CHW = 624
_TAIL_BASE = _NW * _CHW
_TAIL = _NUM_SEL - _TAIL_BASE

_mesh = plsc.VectorSubcoreMesh(
    core_axis_name="c", subcore_axis_name="s",
    num_cores=_NUM_SC_CORES, num_subcores=_NUM_SC_SUBCORES,
)

_col_ty = jax.ShapeDtypeStruct((_NUM_SEL,), jnp.float32)


@functools.partial(
    pl.kernel,
    out_type=[_col_ty] * 6,
    mesh=_mesh,
    scratch_types=[
        pltpu.VMEM((_CHW,), jnp.int32),
        pltpu.VMEM((6, _CHW), jnp.float32),
        pltpu.VMEM((_TAIL,), jnp.int32),
        pltpu.VMEM((6, _TAIL), jnp.float32),
        pltpu.SemaphoreType.DMA,
    ],
    compiler_params=pltpu.CompilerParams(use_tc_tiling_on_sc=False),
)
def _sc_gather(b0, b1, b2, b3, mx, cf, idx_hbm,
               o0, o1, o2, o3, o4, o5,
               idx_v, cols_v, tidx_v, tcols_v, sem):
    wid = lax.axis_index("s") * _NUM_SC_CORES + lax.axis_index("c")
    srcs = (b0, b1, b2, b3, mx, cf)
    outs = (o0, o1, o2, o3, o4, o5)

    base = wid * _CHW
    pltpu.sync_copy(idx_hbm.at[pl.ds(base, _CHW)], idx_v)
    copies = [pltpu.async_copy(src.at[idx_v], cols_v.at[j], sem)
              for j, src in enumerate(srcs)]
    for c in copies:
        c.wait()
    for j, out in enumerate(outs):
        pltpu.sync_copy(cols_v.at[j], out.at[pl.ds(base, _CHW)])

    @pl.when(wid == 0)
    def _tail():
        pltpu.sync_copy(idx_hbm.at[pl.ds(_TAIL_BASE, _TAIL)], tidx_v)
        tcopies = [pltpu.async_copy(src.at[tidx_v], tcols_v.at[j], sem)
                   for j, src in enumerate(srcs)]
        for c in tcopies:
            c.wait()
        for j, out in enumerate(outs):
            pltpu.sync_copy(tcols_v.at[j], out.at[pl.ds(_TAIL_BASE, _TAIL)])


def kernel(idxTensor, boxes, scores):
    maxv, clsf = _cls_reduce(scores[0])
    idx = idxTensor[:, 2]
    cols = _sc_gather(
        boxes[0, 0], boxes[0, 1], boxes[0, 2], boxes[0, 3],
        maxv, clsf, idx,
    )
    concatenated = jnp.stack(cols, axis=1)
    batches = idxTensor[:, 0]
    return (concatenated, batches)


# speedup vs baseline: 2.5459x; 1.0148x over previous
import functools

import jax
import jax.numpy as jnp
from jax import lax
from jax.experimental import pallas as pl
from jax.experimental.pallas import tpu as pltpu
from jax.experimental.pallas import tpu_sc as plsc

_NUM_BOXES = 20000
_NUM_CLASSES = 80
_NUM_SEL = 20000


_BLK = _NUM_BOXES
_GRID = (_NUM_BOXES + _BLK - 1) // _BLK


def _reduce_body(s_ref, maxv_ref, cls_ref):
    s = s_ref[...]
    m = jnp.max(s, axis=0, keepdims=True)
    ids = lax.broadcasted_iota(jnp.int32, s.shape, 0)
    cl = jnp.min(jnp.where(s == m, ids, _NUM_CLASSES), axis=0, keepdims=True)
    maxv_ref[...] = m[0]
    cls_ref[...] = cl.astype(jnp.float32)[0]


def _cls_reduce(scores2d):
    return pl.pallas_call(
        _reduce_body,
        grid=(_GRID,),
        in_specs=[pl.BlockSpec((_NUM_CLASSES, _BLK), lambda i: (0, i))],
        out_specs=[
            pl.BlockSpec((_BLK,), lambda i: (i,)),
            pl.BlockSpec((_BLK,), lambda i: (i,)),
        ],
        out_shape=[
            jax.ShapeDtypeStruct((_NUM_BOXES,), jnp.float32),
            jax.ShapeDtypeStruct((_NUM_BOXES,), jnp.float32),
        ],
    )(scores2d)


_NUM_SC_CORES = 2
_NUM_SC_SUBCORES = 16
_NW = _NUM_SC_CORES * _NUM_SC_SUBCORES
_CHW = 624
_TAIL_BASE = _NW * _CHW
_TAIL = _NUM_SEL - _TAIL_BASE

_mesh = plsc.VectorSubcoreMesh(
    core_axis_name="c", subcore_axis_name="s",
    num_cores=_NUM_SC_CORES, num_subcores=_NUM_SC_SUBCORES,
)

_col_ty = jax.ShapeDtypeStruct((_NUM_SEL,), jnp.float32)


@functools.partial(
    pl.kernel,
    out_type=[_col_ty] * 6,
    mesh=_mesh,
    scratch_types=[
        pltpu.VMEM((_CHW,), jnp.int32),
        pltpu.VMEM((6, _CHW), jnp.float32),
        pltpu.VMEM((_TAIL,), jnp.int32),
        pltpu.VMEM((6, _TAIL), jnp.float32),
        pltpu.SemaphoreType.DMA,
    ],
    compiler_params=pltpu.CompilerParams(use_tc_tiling_on_sc=False),
)
def _sc_gather(b0, b1, b2, b3, mx, cf, idx_hbm,
               o0, o1, o2, o3, o4, o5,
               idx_v, cols_v, tidx_v, tcols_v, sem):
    wid = lax.axis_index("s") * _NUM_SC_CORES + lax.axis_index("c")
    srcs = (b0, b1, b2, b3, mx, cf)
    outs = (o0, o1, o2, o3, o4, o5)

    base = wid * _CHW
    pltpu.sync_copy(idx_hbm.at[pl.ds(base, _CHW)], idx_v)
    copies = [pltpu.async_copy(src.at[idx_v], cols_v.at[j], sem)
              for j, src in enumerate(srcs)]
    for c in copies:
        c.wait()
    for j, out in enumerate(outs):
        pltpu.sync_copy(cols_v.at[j], out.at[pl.ds(base, _CHW)])

    @pl.when(wid == 0)
    def _tail():
        pltpu.sync_copy(idx_hbm.at[pl.ds(_TAIL_BASE, _TAIL)], tidx_v)
        tcopies = [pltpu.async_copy(src.at[tidx_v], tcols_v.at[j], sem)
                   for j, src in enumerate(srcs)]
        for c in tcopies:
            c.wait()
        for j, out in enumerate(outs):
            pltpu.sync_copy(tcols_v.at[j], out.at[pl.ds(_TAIL_BASE, _TAIL)])


def kernel(idxTensor, boxes, scores):
    maxv, clsf = _cls_reduce(scores[0])
    idx = idxTensor[:, 2]
    cols = _sc_gather(
        boxes[0, 0], boxes[0, 1], boxes[0, 2], boxes[0, 3],
        maxv, clsf, idx,
    )
    concatenated = jnp.stack(cols, axis=0).T
    batches = idxTensor[:, 0]
    return (concatenated, batches)
